# baseline (speedup 1.0000x reference)
import functools

import jax
import jax.numpy as jnp
from jax import lax
from jax.experimental import pallas as pl
from jax.experimental.pallas import tpu as pltpu
from jax.experimental.pallas import tpu_sc as plsc

_VOCAB = 1000000
_D = 64
_BATCH = 4096
_SEQ = 200
_TOTAL = _BATCH * _SEQ
_NC = 2
_NS = 16
_NW = _NC * _NS
_PER_W = _TOTAL // _NW
_CHUNK = 128
_NCHUNK = _PER_W // _CHUNK
_M = 8
_K = 4
_SCALE = 8.0

_mesh = plsc.VectorSubcoreMesh(core_axis_name="c", subcore_axis_name="s")


@functools.partial(
    pl.kernel,
    mesh=_mesh,
    out_type=jax.ShapeDtypeStruct((_TOTAL, _D), jnp.float32),
    scratch_types=(
        [pltpu.VMEM((_NCHUNK, _CHUNK), jnp.int32)]
        + [pltpu.VMEM((_CHUNK, _D), jnp.float32)] * _M
        + [pltpu.SemaphoreType.DMA] * (2 * _M)
    ),
    compiler_params=pltpu.CompilerParams(use_tc_tiling_on_sc=False),
)
def _emb_kernel(idx_hbm, table_hbm, out_hbm, idx_v, *rest):
    bufs = rest[:_M]
    gsem = rest[_M:2 * _M]
    ssem = rest[2 * _M:]

    wid = lax.axis_index("s") * _NC + lax.axis_index("c")
    pltpu.sync_copy(idx_hbm.at[pl.ds(wid * _NCHUNK, _NCHUNK)], idx_v)

    out_base = wid * _PER_W

    def fire_gather(chunk, b):
        pltpu.async_copy(table_hbm.at[idx_v.at[chunk]], bufs[b], gsem[b])

    def wait_gather(chunk, b):
        pltpu.make_async_copy(table_hbm.at[idx_v.at[chunk]], bufs[b], gsem[b]).wait()

    def out_slice(chunk):
        return out_hbm.at[pl.ds(out_base + chunk * _CHUNK, _CHUNK)]

    def fire_store(chunk, b):
        pltpu.async_copy(bufs[b], out_slice(chunk), ssem[b])

    def wait_store(chunk, b):
        pltpu.make_async_copy(bufs[b], out_slice(chunk), ssem[b]).wait()

    def scale_buf(buf):
        def row_body(r, carry):
            for c in range(_D // 16):
                buf[r, pl.ds(c * 16, 16)] = buf[r, pl.ds(c * 16, 16)] * _SCALE
            return carry
        lax.fori_loop(0, _CHUNK, row_body, 0, unroll=4)

    for b in range(_K):
        fire_gather(b, b)

    def body(j, carry):
        for b in range(_M):
            c = j * _M + b
            wait_gather(c, b)
            scale_buf(bufs[b])
            fire_store(c, b)
            f = c + _K
            fb = (b + _K) % _M

            @pl.when(f < _NCHUNK)
            def _():
                @pl.when(c >= _K)
                def _():
                    wait_store(c - _K, fb)
                fire_gather(f, fb)
        return carry

    lax.fori_loop(0, _NCHUNK // _M, body, 0)

    for t in range(2 * _K):
        c = _NCHUNK - 2 * _K + t
        wait_store(c, c % _M)


def kernel(x, input_embedding_table):
    idx = x.reshape(_NW * _NCHUNK, _CHUNK).astype(jnp.int32)
    out = _emb_kernel(idx, input_embedding_table)
    return out.reshape(_BATCH, _SEQ, _D)
